# baseline (speedup 1.0000x reference)
import functools
from math import sqrt

import jax
import jax.numpy as jnp
from jax import lax
from jax.experimental import pallas as pl
from jax.experimental.pallas import tpu as pltpu
from jax.experimental.pallas import tpu_sc as plsc

NK = 128
NV = 128
TOPK = 8
NA = 4096
NB = 32
BS = 32
BGRP = 4
NLANE = BGRP * NB
NEG = -1e30


def _scores(key_ref, q_ref, dst_ref):
    cols = []
    for i in range(BGRP):
        cols.append(jnp.dot(key_ref[i], q_ref[i].T,
                            preferred_element_type=jnp.float32,
                            precision=jax.lax.Precision.HIGHEST))
    dst_ref[...] = jnp.concatenate(cols, axis=1) * (1.0 / sqrt(NK))


CH = 512
NCH = NA // CH


def _topk(s_ref, bwn_ref, idx_ref):
    idx = jnp.full((1, NLANE), NA, jnp.int32)
    bw_rows = []
    bi_rows = []
    for _ in range(TOPK):
        m = None
        for c in range(NCH):
            rowc = jax.lax.broadcasted_iota(jnp.int32, (CH, NLANE), 0) + c * CH
            v = s_ref[pl.ds(c * CH, CH), :]
            v = jnp.where(rowc == idx, NEG, v)
            s_ref[pl.ds(c * CH, CH), :] = v
            pm = jnp.max(v, axis=0, keepdims=True)
            m = pm if m is None else jnp.maximum(m, pm)
        iacc = jnp.full((1, NLANE), NA, jnp.int32)
        for c in range(NCH):
            rowc = jax.lax.broadcasted_iota(jnp.int32, (CH, NLANE), 0) + c * CH
            v = s_ref[pl.ds(c * CH, CH), :]
            cand = jnp.where(v == m, rowc, NA)
            iacc = jnp.minimum(iacc, jnp.min(cand, axis=0, keepdims=True))
        idx = iacc
        bw_rows.append(m)
        bi_rows.append(idx)

    bw = jnp.concatenate(bw_rows, axis=0)
    bi = jnp.concatenate(bi_rows, axis=0)

    mean = jnp.mean(bw, axis=0, keepdims=True)
    var = jnp.sum((bw - mean) ** 2, axis=0, keepdims=True) / (TOPK - 1)
    std = jnp.sqrt(var) + 1e-8
    std = std / (1.0 + std)
    z = (bw - mean) / std
    z = z - jnp.max(z, axis=0, keepdims=True)
    e = jnp.exp(z)
    bwn = e / jnp.sum(e, axis=0, keepdims=True)

    bwn_ref[...] = bwn.reshape(TOPK, BGRP, NB).transpose(1, 0, 2)
    idx_ref[...] = bi.reshape(TOPK, BGRP, NB).transpose(1, 0, 2)


def _tc_body(key_ref, q_ref, bwn_ref, idx_ref, s_a, s_b):
    _scores(key_ref, q_ref, s_a)
    _topk(s_b, bwn_ref, idx_ref)
    s_b[...] = s_a[...]


def _tc_topk(key3, query3):
    ngrp = BS // BGRP
    out_shapes = (
        jax.ShapeDtypeStruct((BS, TOPK, NB), jnp.float32),
        jax.ShapeDtypeStruct((BS, TOPK, NB), jnp.int32),
    )
    return pl.pallas_call(
        _tc_body,
        grid=(ngrp + 1,),
        in_specs=[
            pl.BlockSpec((BGRP, NA, NK), lambda j: (jnp.minimum(j, ngrp - 1), 0, 0)),
            pl.BlockSpec((BGRP, NB, NK), lambda j: (jnp.minimum(j, ngrp - 1), 0, 0)),
        ],
        out_specs=(
            pl.BlockSpec((BGRP, TOPK, NB), lambda j: (jnp.maximum(j - 1, 0), 0, 0)),
            pl.BlockSpec((BGRP, TOPK, NB), lambda j: (jnp.maximum(j - 1, 0), 0, 0)),
        ),
        out_shape=out_shapes,
        scratch_shapes=[
            pltpu.VMEM((NA, NLANE), jnp.float32),
            pltpu.VMEM((NA, NLANE), jnp.float32),
        ],
        compiler_params=pltpu.CompilerParams(
            dimension_semantics=("arbitrary",),
        ),
    )(key3, query3)


def _lane_gather(x, idx):
    return lax.gather(
        x, idx[:, None],
        dimension_numbers=lax.GatherDimensionNumbers(
            offset_dims=(), collapsed_slice_dims=(0,), start_index_map=(0,)),
        slice_sizes=(1,),
        mode=lax.GatherScatterMode.PROMISE_IN_BOUNDS)


def _lane_total(x):
    lane = lax.iota(jnp.int32, 16)
    for sh in (8, 4, 2, 1):
        x = x + _lane_gather(x, lax.bitwise_xor(lane, sh))
    return x


def _rsqrt_nr(x):
    i = lax.bitcast_convert_type(x, jnp.int32)
    i = 0x5F3759DF - lax.shift_right_arithmetic(i, 1)
    y = lax.bitcast_convert_type(i, jnp.float32)
    for _ in range(3):
        y = y * (1.5 - 0.5 * x * y * y)
    return y


def _sc_body(value_hbm, bi_hbm, bwn_hbm, out_hbm, idx_v, w_v, g_v, o_v, sem):
    b = lax.axis_index("s") * 2 + lax.axis_index("c")
    pltpu.sync_copy(bi_hbm.at[b], idx_v)
    pltpu.sync_copy(bwn_hbm.at[b], w_v)
    c0 = pltpu.async_copy(value_hbm.at[b].at[idx_v.at[0]],
                          g_v.at[pl.ds(0, 128)], sem)
    c1 = pltpu.async_copy(value_hbm.at[b].at[idx_v.at[1]],
                          g_v.at[pl.ds(128, 128)], sem)
    c0.wait()
    c1.wait()

    def body(nb, carry):
        wts = []
        for t in range(TOPK):
            wts.append(w_v[t * NB + nb, :])
        chunks = []
        ss = jnp.zeros((16,), jnp.float32)
        for c in range(NV // 16):
            acc = jnp.zeros((16,), jnp.float32)
            for t in range(TOPK):
                acc = acc + wts[t] * g_v[t * NB + nb, pl.ds(c * 16, 16)]
            chunks.append(acc)
            ss = ss + acc * acc
        tot = _lane_total(ss)
        y = _rsqrt_nr(jnp.maximum(tot, 1e-24))
        for c in range(NV // 16):
            o_v[nb, pl.ds(c * 16, 16)] = chunks[c] * y
        return carry

    lax.fori_loop(0, NB, body, 0)
    pltpu.sync_copy(o_v, out_hbm.at[b])


def _sc_combine(value, bi2, bwn_splat):
    mesh = plsc.VectorSubcoreMesh(core_axis_name="c", subcore_axis_name="s")
    f = functools.partial(
        pl.kernel,
        mesh=mesh,
        out_type=jax.ShapeDtypeStruct((BS, NB, NV), jnp.float32),
        scratch_types=[
            pltpu.VMEM((2, 128), jnp.int32),
            pltpu.VMEM((TOPK * NB, 16), jnp.float32),
            pltpu.VMEM((TOPK * NB, NV), jnp.float32),
            pltpu.VMEM((NB, NV), jnp.float32),
            pltpu.SemaphoreType.DMA,
        ],
    )(_sc_body)
    return f(value, bi2, bwn_splat)


def kernel(key, query, value):
    bwn, bi = _tc_topk(key.reshape(BS, NA, NK),
                       query.reshape(BS, NB, NK))
    bwn_splat = jnp.broadcast_to(
        bwn.reshape(BS, TOPK * NB, 1), (BS, TOPK * NB, 16))
    rows = _sc_combine(value,
                       bi.reshape(BS, 2, 128),
                       bwn_splat)
    vals = jnp.swapaxes(rows, 1, 2)
    return (vals, bwn, bi)


# speedup vs baseline: 1.0183x; 1.0183x over previous
import functools
from math import sqrt

import jax
import jax.numpy as jnp
from jax import lax
from jax.experimental import pallas as pl
from jax.experimental.pallas import tpu as pltpu
from jax.experimental.pallas import tpu_sc as plsc

NK = 128
NV = 128
TOPK = 8
NA = 4096
NB = 32
BS = 32
BGRP = 4
NLANE = BGRP * NB
NEG = -1e30


def _scores(key_ref, q_ref, dst_ref):
    cols = []
    for i in range(BGRP):
        cols.append(jnp.dot(key_ref[i], q_ref[i].T,
                            preferred_element_type=jnp.float32,
                            precision=jax.lax.Precision.HIGHEST))
    dst_ref[...] = jnp.concatenate(cols, axis=1) * (1.0 / sqrt(NK))


CH = 512
NCH = NA // CH


def _topk(s_ref, bwn_ref, idx_ref):
    idx = jnp.full((1, NLANE), NA, jnp.int32)
    bw_rows = []
    bi_rows = []
    for _ in range(TOPK):
        m = None
        for c in range(NCH):
            rowc = jax.lax.broadcasted_iota(jnp.int32, (CH, NLANE), 0) + c * CH
            v = s_ref[pl.ds(c * CH, CH), :]
            v = jnp.where(rowc == idx, NEG, v)
            s_ref[pl.ds(c * CH, CH), :] = v
            pm = jnp.max(v, axis=0, keepdims=True)
            m = pm if m is None else jnp.maximum(m, pm)
        iacc = jnp.full((1, NLANE), NA, jnp.int32)
        for c in range(NCH):
            rowc = jax.lax.broadcasted_iota(jnp.int32, (CH, NLANE), 0) + c * CH
            v = s_ref[pl.ds(c * CH, CH), :]
            cand = jnp.where(v == m, rowc, NA)
            iacc = jnp.minimum(iacc, jnp.min(cand, axis=0, keepdims=True))
        idx = iacc
        bw_rows.append(m)
        bi_rows.append(idx)

    bw = jnp.concatenate(bw_rows, axis=0)
    bi = jnp.concatenate(bi_rows, axis=0)

    mean = jnp.mean(bw, axis=0, keepdims=True)
    var = jnp.sum((bw - mean) ** 2, axis=0, keepdims=True) / (TOPK - 1)
    std = jnp.sqrt(var) + 1e-8
    std = std / (1.0 + std)
    z = (bw - mean) / std
    z = z - jnp.max(z, axis=0, keepdims=True)
    e = jnp.exp(z)
    bwn = e / jnp.sum(e, axis=0, keepdims=True)

    bwn_ref[...] = bwn.reshape(TOPK, BGRP, NB).transpose(1, 0, 2)
    idx_ref[...] = bi.reshape(TOPK, BGRP, NB).transpose(1, 0, 2)


def _tc_body(key_ref, q_ref, bwn_ref, idx_ref, s_a, s_b):
    _scores(key_ref, q_ref, s_a)
    _topk(s_b, bwn_ref, idx_ref)
    s_b[...] = s_a[...]


def _tc_topk(key3, query3):
    ngrp = BS // BGRP
    out_shapes = (
        jax.ShapeDtypeStruct((BS, TOPK, NB), jnp.float32),
        jax.ShapeDtypeStruct((BS, TOPK, NB), jnp.int32),
    )
    return pl.pallas_call(
        _tc_body,
        grid=(ngrp + 1,),
        in_specs=[
            pl.BlockSpec((BGRP, NA, NK), lambda j: (jnp.minimum(j, ngrp - 1), 0, 0)),
            pl.BlockSpec((BGRP, NB, NK), lambda j: (jnp.minimum(j, ngrp - 1), 0, 0)),
        ],
        out_specs=(
            pl.BlockSpec((BGRP, TOPK, NB), lambda j: (jnp.maximum(j - 1, 0), 0, 0)),
            pl.BlockSpec((BGRP, TOPK, NB), lambda j: (jnp.maximum(j - 1, 0), 0, 0)),
        ),
        out_shape=out_shapes,
        scratch_shapes=[
            pltpu.VMEM((NA, NLANE), jnp.float32),
            pltpu.VMEM((NA, NLANE), jnp.float32),
        ],
        compiler_params=pltpu.CompilerParams(
            dimension_semantics=("arbitrary",),
        ),
    )(key3, query3)


def _lane_gather(x, idx):
    return lax.gather(
        x, idx[:, None],
        dimension_numbers=lax.GatherDimensionNumbers(
            offset_dims=(), collapsed_slice_dims=(0,), start_index_map=(0,)),
        slice_sizes=(1,),
        mode=lax.GatherScatterMode.PROMISE_IN_BOUNDS)


def _lane_total(x):
    lane = lax.iota(jnp.int32, 16)
    for sh in (8, 4, 2, 1):
        x = x + _lane_gather(x, lax.bitwise_xor(lane, sh))
    return x


def _rsqrt_nr(x):
    i = lax.bitcast_convert_type(x, jnp.int32)
    i = 0x5F3759DF - lax.shift_right_arithmetic(i, 1)
    y = lax.bitcast_convert_type(i, jnp.float32)
    for _ in range(3):
        y = y * (1.5 - 0.5 * x * y * y)
    return y


def _sc_body(value_hbm, bi_hbm, bwn_hbm, out_hbm, idx_v, w_v, g_v, o_v, sem):
    b = lax.axis_index("s") * 2 + lax.axis_index("c")
    pltpu.sync_copy(bi_hbm.at[b], idx_v)
    pltpu.sync_copy(bwn_hbm.at[b], w_v)
    c0 = pltpu.async_copy(value_hbm.at[b].at[idx_v.at[0]],
                          g_v.at[pl.ds(0, 128)], sem)
    c1 = pltpu.async_copy(value_hbm.at[b].at[idx_v.at[1]],
                          g_v.at[pl.ds(128, 128)], sem)
    c0.wait()
    c1.wait()
    lane = lax.iota(jnp.int32, 16)

    def body(nb, carry):
        zero = jnp.zeros((16,), jnp.int32)
        wts = []
        for t in range(TOPK):
            wrow = w_v[2 * t + nb // 16, :]
            wts.append(_lane_gather(wrow, zero + lax.rem(nb, 16)))
        chunks = []
        ss = jnp.zeros((16,), jnp.float32)
        for c in range(NV // 16):
            acc = jnp.zeros((16,), jnp.float32)
            for t in range(TOPK):
                acc = acc + wts[t] * g_v[t * NB + nb, pl.ds(c * 16, 16)]
            chunks.append(acc)
            ss = ss + acc * acc
        tot = _lane_total(ss)
        y = _rsqrt_nr(jnp.maximum(tot, 1e-24))
        for c in range(NV // 16):
            o_v[nb, pl.ds(c * 16, 16)] = chunks[c] * y
        return carry

    lax.fori_loop(0, NB, body, 0)
    pltpu.sync_copy(o_v, out_hbm.at[b])


def _sc_combine(value, bi2, bwn2):
    mesh = plsc.VectorSubcoreMesh(core_axis_name="c", subcore_axis_name="s")
    f = functools.partial(
        pl.kernel,
        mesh=mesh,
        out_type=jax.ShapeDtypeStruct((BS, NB, NV), jnp.float32),
        scratch_types=[
            pltpu.VMEM((2, 128), jnp.int32),
            pltpu.VMEM((16, 16), jnp.float32),
            pltpu.VMEM((TOPK * NB, NV), jnp.float32),
            pltpu.VMEM((NB, NV), jnp.float32),
            pltpu.SemaphoreType.DMA,
        ],
    )(_sc_body)
    return f(value, bi2, bwn2)


def kernel(key, query, value):
    bwn, bi = _tc_topk(key.reshape(BS, NA, NK),
                       query.reshape(BS, NB, NK))
    rows = _sc_combine(value,
                       bi.reshape(BS, 2, 128),
                       bwn.reshape(BS, 16, 16))
    vals = jnp.swapaxes(rows, 1, 2)
    return (vals, bwn, bi)
